# Initial kernel scaffold; baseline (speedup 1.0000x reference)
import jax
import jax.numpy as jnp
from jax.experimental import pallas as pl


def kernel(inputs, adj_lst, mask_index, W, a, b):
    raise NotImplementedError("write your pallas kernel here")


# speedup vs baseline: 10.6692x; 10.6692x over previous
import functools

import jax
import jax.numpy as jnp
from jax import lax
from jax.experimental import pallas as pl
from jax.experimental.pallas import tpu as pltpu
from jax.experimental.pallas import tpu_sc as plsc

_NC, _NS, _L = 2, 16, 16


def _tc_transform(x, Wcat, A):
    N, D = x.shape
    HD = Wcat.shape[1]
    BN = 400

    def body(x_ref, w_ref, a_ref, t_ref, s_ref):
        t = jnp.dot(x_ref[...], w_ref[...], preferred_element_type=jnp.float32)
        t_ref[...] = t
        s_ref[...] = jnp.dot(t, a_ref[...], preferred_element_type=jnp.float32)

    return pl.pallas_call(
        body,
        grid=(N // BN,),
        in_specs=[
            pl.BlockSpec((BN, D), lambda i: (i, 0)),
            pl.BlockSpec((D, HD), lambda i: (0, 0)),
            pl.BlockSpec((HD, 128), lambda i: (0, 0)),
        ],
        out_specs=[
            pl.BlockSpec((BN, HD), lambda i: (i, 0)),
            pl.BlockSpec((BN, 128), lambda i: (i, 0)),
        ],
        out_shape=[
            jax.ShapeDtypeStruct((N, HD), jnp.float32),
            jax.ShapeDtypeStruct((N, 128), jnp.float32),
        ],
    )(x, Wcat, A)


def _make_sc_gat(N, K, H, D, tile_n):
    HD = H * D
    NW = _NC * _NS
    NPAD = NW * tile_n
    JV = D // _L
    UK = K // _L
    mesh = plsc.VectorSubcoreMesh(
        core_axis_name="c", subcore_axis_name="s",
        num_cores=_NC, num_subcores=_NS)

    @functools.partial(
        pl.kernel,
        out_type=jax.ShapeDtypeStruct((NPAD * D,), jnp.float32),
        mesh=mesh,
        compiler_params=pltpu.CompilerParams(needs_layout_passes=False),
        scratch_types=[
            pltpu.VMEM((N * H,), jnp.float32),
            pltpu.VMEM((tile_n * K,), jnp.int32),
            pltpu.VMEM((2, K), jnp.int32),
            pltpu.VMEM((2, K, HD), jnp.float32),
            pltpu.VMEM((2, D), jnp.float32),
            pltpu.VMEM((D,), jnp.float32),
            pltpu.SemaphoreType.DMA,
            pltpu.SemaphoreType.DMA,
            pltpu.SemaphoreType.DMA,
            pltpu.SemaphoreType.DMA,
        ],
    )
    def sc_gat(t_hbm, s_hbm, adj_hbm, bb_hbm, out_hbm,
               s_tab, adj_buf, idx_st, rows, outb, bbv,
               sem0, sem1, osem0, osem1):
        wid = lax.axis_index("s") * _NC + lax.axis_index("c")
        base = wid * tile_n
        pltpu.sync_copy(s_hbm, s_tab)
        pltpu.sync_copy(adj_hbm.at[pl.ds(base * K, tile_n * K)], adj_buf)
        pltpu.sync_copy(bb_hbm, bbv)
        sems = (sem0, sem1)
        osems = (osem0, osem1)

        def prep(g, slot):
            for u in range(UK):
                av = adj_buf[pl.ds(g * K + u * _L, _L)]
                idx_st[slot, pl.ds(u * _L, _L)] = jnp.where(
                    av < 0, jnp.int32(N - 1), av)

        def fire(slot):
            pltpu.make_async_copy(
                t_hbm.at[idx_st.at[slot]], rows.at[slot], sems[slot]).start()

        def wait(slot):
            pltpu.make_async_copy(
                t_hbm.at[idx_st.at[slot]], rows.at[slot], sems[slot]).wait()

        def out_copy(g, slot):
            return pltpu.make_async_copy(
                outb.at[slot], out_hbm.at[pl.ds((base + g) * D, D)],
                osems[slot])

        def compute(g, slot):
            coefs = []
            for h in range(H):
                lv = []
                for u in range(UK):
                    av = adj_buf[pl.ds(g * K + u * _L, _L)]
                    iv = jnp.where(av < 0, jnp.int32(N - 1), av)
                    gathered = plsc.load_gather(s_tab, [iv * H + h])
                    lv.append(jnp.where(av < 0, jnp.float32(-1e9), gathered))
                m = jnp.max(lv[0])
                for u in range(1, UK):
                    m = jnp.maximum(m, jnp.max(lv[u]))
                ev = [jnp.exp(v - m) for v in lv]
                tot = jnp.sum(ev[0])
                for u in range(1, UK):
                    tot = tot + jnp.sum(ev[u])
                inv = jnp.broadcast_to(1.0 / H, (_L,)) / jnp.broadcast_to(
                    tot, (_L,))
                coefs.append([e * inv for e in ev])
            acc = [bbv[pl.ds(j * _L, _L)] for j in range(JV)]
            for h in range(H):
                for u in range(UK):
                    cv = coefs[h][u]
                    for kl in range(_L):
                        kk = u * _L + kl
                        c = cv[kl]
                        for j in range(JV):
                            acc[j] = acc[j] + c * rows[slot, kk,
                                                       pl.ds(h * D + j * _L, _L)]
            @pl.when(g >= 2)
            def _():
                out_copy(g - 2, slot).wait()
            for j in range(JV):
                outb[slot, pl.ds(j * _L, _L)] = jnp.maximum(acc[j], 0.0)
            out_copy(g, slot).start()

        prep(0, 0)
        fire(0)
        prep(1, 1)
        fire(1)

        @pl.loop(0, tile_n, step=2)
        def _(g):
            wait(0)
            compute(g, 0)
            prep(jnp.minimum(g + 2, tile_n - 1), 0)
            fire(0)
            wait(1)
            compute(g + 1, 1)
            prep(jnp.minimum(g + 3, tile_n - 1), 1)
            fire(1)

        wait(0)
        wait(1)
        out_copy(tile_n - 2, 0).wait()
        out_copy(tile_n - 1, 1).wait()

    return sc_gat


def kernel(inputs, adj_lst, mask_index, W, a, b):
    N, D = inputs.shape
    H = W.shape[0]
    K = adj_lst.shape[1]
    HD = H * D

    Wcat = jnp.transpose(W, (1, 0, 2)).reshape(D, HD)
    avec = a[:, :, 0].reshape(HD)
    A = jnp.zeros((HD, 128), jnp.float32).at[
        jnp.arange(HD), jnp.repeat(jnp.arange(H), D)].set(avec)
    bbar = jnp.mean(b, axis=0)

    adj32 = adj_lst.astype(jnp.int32)
    mi = jnp.asarray(mask_index, jnp.int32)
    adj_enc = jnp.where(adj32 == mi, jnp.int32(-1),
                        jnp.minimum(adj32, jnp.int32(N - 1)))

    NW = _NC * _NS
    tile_n = (-(-N // NW) + 7) // 8 * 8
    NPAD = NW * tile_n
    adj_enc = jnp.concatenate(
        [adj_enc, jnp.full((NPAD - N, K), -1, jnp.int32)]).reshape(-1)

    t, s_pad = _tc_transform(inputs, Wcat, A)
    s_flat = s_pad[:, :H].reshape(-1)

    out_flat = _make_sc_gat(N, K, H, D, tile_n)(t, s_flat, adj_enc, bbar)
    return out_flat.reshape(NPAD, D)[:N]


# speedup vs baseline: 12.8189x; 1.2015x over previous
import functools

import jax
import jax.numpy as jnp
from jax import lax
from jax.experimental import pallas as pl
from jax.experimental.pallas import tpu as pltpu
from jax.experimental.pallas import tpu_sc as plsc

_NC, _NS, _L = 2, 16, 16
_RW = 528


def _tc_transform(x, Wcat, A):
    N, D = x.shape
    HD = Wcat.shape[1]
    BN = 400

    def body(x_ref, w_ref, a_ref, o_ref):
        t = jnp.dot(x_ref[...], w_ref[...], preferred_element_type=jnp.float32)
        s = jnp.dot(t, a_ref[...], preferred_element_type=jnp.float32)
        o_ref[...] = jnp.concatenate([t, s], axis=1)

    return pl.pallas_call(
        body,
        grid=(N // BN,),
        in_specs=[
            pl.BlockSpec((BN, D), lambda i: (i, 0)),
            pl.BlockSpec((D, HD), lambda i: (0, 0)),
            pl.BlockSpec((HD, _RW - HD), lambda i: (0, 0)),
        ],
        out_specs=pl.BlockSpec((BN, _RW), lambda i: (i, 0)),
        out_shape=jax.ShapeDtypeStruct((N, _RW), jnp.float32),
    )(x, Wcat, A)


def _make_sc_gat(N, K, H, D, tile_n):
    HD = H * D
    NW = _NC * _NS
    NPAD = NW * tile_n
    JV = D // _L
    UK = K // _L
    mesh = plsc.VectorSubcoreMesh(
        core_axis_name="c", subcore_axis_name="s",
        num_cores=_NC, num_subcores=_NS)

    @functools.partial(
        pl.kernel,
        out_type=jax.ShapeDtypeStruct((NPAD * D,), jnp.float32),
        mesh=mesh,
        compiler_params=pltpu.CompilerParams(
            needs_layout_passes=False, use_tc_tiling_on_sc=False),
        scratch_types=[
            pltpu.VMEM((tile_n * K,), jnp.int32),
            pltpu.VMEM((2, 2 * K), jnp.int32),
            pltpu.VMEM((2, 2 * K, _RW), jnp.float32),
            pltpu.VMEM((2, D), jnp.float32),
            pltpu.VMEM((H * K,), jnp.float32),
            pltpu.VMEM((D,), jnp.float32),
            pltpu.SemaphoreType.DMA,
            pltpu.SemaphoreType.DMA,
            pltpu.SemaphoreType.DMA,
            pltpu.SemaphoreType.DMA,
        ],
    )
    def sc_gat(t_hbm, adj_hbm, bb_hbm, out_hbm,
               adj_buf, idx_st, rows, outb, cbuf, bbv,
               sem0, sem1, osem0, osem1):
        wid = lax.axis_index("s") * _NC + lax.axis_index("c")
        base = wid * tile_n
        pltpu.sync_copy(adj_hbm.at[pl.ds(base * K, tile_n * K)], adj_buf)
        pltpu.sync_copy(bb_hbm, bbv)
        sems = (sem0, sem1)
        osems = (osem0, osem1)
        lanes = lax.iota(jnp.int32, _L)

        def prep2(gp, slot):
            for nd in range(2):
                for u in range(UK):
                    av = adj_buf[pl.ds((gp + nd) * K + u * _L, _L)]
                    idx_st[slot, pl.ds(nd * K + u * _L, _L)] = jnp.where(
                        av < 0, jnp.int32(N - 1), av)

        def fire(slot):
            pltpu.make_async_copy(
                t_hbm.at[idx_st.at[slot]], rows.at[slot], sems[slot]).start()

        def wait(slot):
            pltpu.make_async_copy(
                t_hbm.at[idx_st.at[slot]], rows.at[slot], sems[slot]).wait()

        def out_copy(g, slot):
            return pltpu.make_async_copy(
                outb.at[slot], out_hbm.at[pl.ds((base + g) * D, D)],
                osems[slot])

        def compute(g, slot, sub):
            for h in range(H):
                lv = []
                for u in range(UK):
                    av = adj_buf[pl.ds(g * K + u * _L, _L)]
                    kvec = jnp.full((_L,), sub * K + u * _L, jnp.int32) + lanes
                    cvec = jnp.full((_L,), HD + h, jnp.int32)
                    svec = jnp.full((_L,), slot, jnp.int32)
                    gathered = plsc.load_gather(rows, [svec, kvec, cvec])
                    lv.append(jnp.where(av < 0, jnp.float32(-1e9), gathered))
                m = jnp.max(lv[0])
                for u in range(1, UK):
                    m = jnp.maximum(m, jnp.max(lv[u]))
                ev = [jnp.exp(v - m) for v in lv]
                tot = jnp.sum(ev[0])
                for u in range(1, UK):
                    tot = tot + jnp.sum(ev[u])
                inv = jnp.broadcast_to(1.0 / H, (_L,)) / jnp.broadcast_to(
                    tot, (_L,))
                for u in range(UK):
                    cbuf[pl.ds(h * K + u * _L, _L)] = ev[u] * inv
            acc0 = tuple(bbv[pl.ds(j * _L, _L)] for j in range(JV))

            @pl.loop(0, H, init_carry=acc0)
            def acc_loop(h, acc):
                acc = list(acc)
                for u in range(UK):
                    cv = cbuf[pl.ds(h * K + u * _L, _L)]
                    for kl in range(_L):
                        kk = sub * K + u * _L + kl
                        c = cv[kl]
                        for j in range(JV):
                            acc[j] = acc[j] + c * rows[slot, kk,
                                                       pl.ds(h * D + j * _L,
                                                             _L)]
                return tuple(acc)

            acc = acc_loop
            @pl.when(g >= 2)
            def _():
                out_copy(g - 2, sub).wait()
            for j in range(JV):
                outb[sub, pl.ds(j * _L, _L)] = jnp.maximum(acc[j], 0.0)
            out_copy(g, sub).start()

        prep2(0, 0)
        fire(0)
        prep2(2, 1)
        fire(1)

        @pl.loop(0, tile_n, step=4)
        def _(g):
            wait(0)
            compute(g, 0, 0)
            compute(g + 1, 0, 1)
            prep2(jnp.minimum(g + 4, tile_n - 2), 0)
            fire(0)
            wait(1)
            compute(g + 2, 1, 0)
            compute(g + 3, 1, 1)
            prep2(jnp.minimum(g + 6, tile_n - 2), 1)
            fire(1)

        wait(0)
        wait(1)
        out_copy(tile_n - 2, 0).wait()
        out_copy(tile_n - 1, 1).wait()

    return sc_gat


def kernel(inputs, adj_lst, mask_index, W, a, b):
    N, D = inputs.shape
    H = W.shape[0]
    K = adj_lst.shape[1]
    HD = H * D

    Wcat = jnp.transpose(W, (1, 0, 2)).reshape(D, HD)
    avec = a[:, :, 0].reshape(HD)
    A = jnp.zeros((HD, _RW - HD), jnp.float32).at[
        jnp.arange(HD), jnp.repeat(jnp.arange(H), D)].set(avec)
    bbar = jnp.mean(b, axis=0)

    adj32 = adj_lst.astype(jnp.int32)
    mi = jnp.asarray(mask_index, jnp.int32)
    adj_enc = jnp.where(adj32 == mi, jnp.int32(-1),
                        jnp.minimum(adj32, jnp.int32(N - 1)))

    NW = _NC * _NS
    tile_n = (-(-N // NW) + 7) // 8 * 8
    NPAD = NW * tile_n
    adj_enc = jnp.concatenate(
        [adj_enc, jnp.full((NPAD - N, K), -1, jnp.int32)]).reshape(-1)

    taug = _tc_transform(inputs, Wcat, A)
    out_flat = _make_sc_gat(N, K, H, D, tile_n)(taug, adj_enc, bbar)
    return out_flat.reshape(NPAD, D)[:N]
